# baseline (speedup 1.0000x reference)
import functools

import jax
import jax.numpy as jnp
from jax import lax
from jax.experimental import pallas as pl
from jax.experimental.pallas import tpu as pltpu
from jax.experimental.pallas import tpu_sc as plsc

N_NODES = 10000
NODE_DIM = 128
N_EDGES = 320000
MAX_DEG = 512

NC = 2
NS = 16
NW = NC * NS

NPAD = 10240
CPT = NPAD // NS
EPT = N_EDGES // NS
IDR = 125
IDW = 160

BLK = 80
NBLK = N_NODES // BLK
MAXB = (NBLK + NW - 1) // NW

_mesh = plsc.VectorSubcoreMesh(core_axis_name="c", subcore_axis_name="s")


@functools.partial(
    pl.kernel,
    out_type=(
        jax.ShapeDtypeStruct((NPAD,), jnp.int32),
        jax.ShapeDtypeStruct((NPAD,), jnp.int32),
    ),
    mesh=_mesh,
    scratch_types=[
        pltpu.VMEM((IDR, IDW), jnp.int32),
        pltpu.VMEM((NPAD,), jnp.int32),
        pltpu.VMEM((CPT,), jnp.int32),
        pltpu.VMEM((CPT,), jnp.int32),
        pltpu.VMEM_SHARED((NS * NPAD,), jnp.int32),
        pltpu.SemaphoreType.DMA,
    ],
    compiler_params=pltpu.CompilerParams(needs_layout_passes=False),
)
def _degrees(edge_hbm, dego_hbm, degi_hbm, idx_v, hist_v, acc_v, tmp_v,
             stage_sh, sem):
    c = lax.axis_index("c")
    s = lax.axis_index("s")
    ones16 = jnp.full((16,), 1, jnp.int32)

    hload = pltpu.async_copy(edge_hbm.at[c, s], idx_v, sem)

    def zrow(r, carry):
        hist_v[pl.ds(r * 16, 16)] = jnp.zeros((16,), jnp.int32)
        return carry

    lax.fori_loop(0, NPAD // 16, zrow, 0)
    hload.wait()

    def hrow(r, carry):
        for j in range(IDW // 16):
            iv = idx_v[r, pl.ds(j * 16, 16)]
            plsc.addupdate_scatter(hist_v, [iv], ones16)
        return carry

    lax.fori_loop(0, IDR, hrow, 0)

    pltpu.sync_copy(hist_v, stage_sh.at[pl.ds(s * NPAD, NPAD)])
    plsc.subcore_barrier()

    pltpu.sync_copy(stage_sh.at[pl.ds(s * CPT, CPT)], acc_v)

    def radd(r, carry):
        pltpu.sync_copy(stage_sh.at[pl.ds(r * NPAD + s * CPT, CPT)], tmp_v)
        for j in range(CPT // 16):
            sl = pl.ds(j * 16, 16)
            acc_v[sl] = acc_v[sl] + tmp_v[sl]
        return carry

    lax.fori_loop(1, NS, radd, 0)

    for j in range(CPT // 16):
        sl = pl.ds(j * 16, 16)
        acc_v[sl] = jnp.minimum(acc_v[sl], MAX_DEG - 1)

    @pl.when(c == 0)
    def _():
        pltpu.sync_copy(acc_v, dego_hbm.at[pl.ds(s * CPT, CPT)])

    @pl.when(c == 1)
    def _():
        pltpu.sync_copy(acc_v, degi_hbm.at[pl.ds(s * CPT, CPT)])


@functools.partial(
    pl.kernel,
    out_type=jax.ShapeDtypeStruct((N_NODES, NODE_DIM), jnp.float32),
    mesh=_mesh,
    scratch_types=[
        pltpu.VMEM((BLK,), jnp.int32),
        pltpu.VMEM((BLK,), jnp.int32),
        pltpu.VMEM((BLK, NODE_DIM), jnp.float32),
        pltpu.VMEM((BLK, NODE_DIM), jnp.float32),
        pltpu.VMEM((BLK, NODE_DIM), jnp.float32),
        pltpu.SemaphoreType.DMA,
        pltpu.SemaphoreType.DMA,
        pltpu.SemaphoreType.DMA,
    ],
)
def _encode(x_hbm, dego_hbm, degi_hbm, zin_hbm, zout_hbm, out_hbm,
            degi_v, dego_v, acc_v, zi_v, zo_v, semx, semi, semo):
    c = lax.axis_index("c")
    s = lax.axis_index("s")
    w = s * NC + c
    for k in range(MAXB):
        b = k * NW + w

        @pl.when(b < NBLK)
        def _():
            base = b * BLK
            hx = pltpu.async_copy(x_hbm.at[pl.ds(base, BLK), :], acc_v, semx)
            pltpu.sync_copy(degi_hbm.at[pl.ds(base, BLK)], degi_v)
            pltpu.sync_copy(dego_hbm.at[pl.ds(base, BLK)], dego_v)
            hi = pltpu.async_copy(zin_hbm.at[degi_v], zi_v, semi)
            ho = pltpu.async_copy(zout_hbm.at[dego_v], zo_v, semo)
            hx.wait()
            hi.wait()
            ho.wait()

            def row_add(r, carry):
                for jj in range(NODE_DIM // 16):
                    sl = pl.ds(jj * 16, 16)
                    acc_v[r, sl] = acc_v[r, sl] + zi_v[r, sl] + zo_v[r, sl]
                return carry

            lax.fori_loop(0, BLK, row_add, 0)
            pltpu.sync_copy(acc_v, out_hbm.at[pl.ds(base, BLK), :])


def kernel(x, edge_index, z_in, z_out):
    ei = edge_index.astype(jnp.int32).reshape(2, NS, IDR, IDW)
    dego, degi = _degrees(ei)
    return _encode(x, dego, degi, z_in, z_out)


# speedup vs baseline: 1.0122x; 1.0122x over previous
import functools

import jax
import jax.numpy as jnp
from jax import lax
from jax.experimental import pallas as pl
from jax.experimental.pallas import tpu as pltpu
from jax.experimental.pallas import tpu_sc as plsc

N_NODES = 10000
NODE_DIM = 128
N_EDGES = 320000
MAX_DEG = 512

NC = 2
NS = 16
NW = NC * NS

NPAD = 10240
CPT = NPAD // NS
ROWW = 100
NROWS = N_EDGES // ROWW
RPT = NROWS // NS
FIRE = 10

BLK = 80
NBLK = N_NODES // BLK
MAXB = (NBLK + NW - 1) // NW

_mesh = plsc.VectorSubcoreMesh(core_axis_name="c", subcore_axis_name="s")


@functools.partial(
    pl.kernel,
    out_type=(
        jax.ShapeDtypeStruct((NPAD,), jnp.int32),
        jax.ShapeDtypeStruct((NPAD,), jnp.int32),
    ),
    mesh=_mesh,
    scratch_types=[
        pltpu.VMEM((RPT, ROWW), jnp.int32),
        pltpu.VMEM((112,), jnp.int32),
        pltpu.VMEM((CPT,), jnp.int32),
        pltpu.VMEM_SHARED((NPAD,), jnp.int32),
        pltpu.SemaphoreType.DMA,
    ],
)
def _degrees(edge_hbm, dego_hbm, degi_hbm, idx_v, ones_v, buf_v, hist_sh, sem):
    c = lax.axis_index("c")
    s = lax.axis_index("s")
    for j in range(112 // 16):
        ones_v[pl.ds(j * 16, 16)] = jnp.full((16,), 1, jnp.int32)
    for j in range(CPT // 16):
        buf_v[pl.ds(j * 16, 16)] = jnp.zeros((16,), jnp.int32)
    pltpu.sync_copy(buf_v, hist_sh.at[pl.ds(s * CPT, CPT)])
    pltpu.sync_copy(edge_hbm.at[c, s], idx_v)
    plsc.subcore_barrier()

    def fire_drain(g, carry):
        hs = []
        for i in range(FIRE):
            j = g * FIRE + i
            hs.append(
                pltpu.async_copy(
                    ones_v.at[pl.ds(0, ROWW)],
                    hist_sh.at[idx_v.at[j]],
                    sem,
                    add=True,
                )
            )
        for h in hs:
            h.wait()
        return carry

    lax.fori_loop(0, RPT // FIRE, fire_drain, 0)
    plsc.subcore_barrier()

    pltpu.sync_copy(hist_sh.at[pl.ds(s * CPT, CPT)], buf_v)
    for j in range(CPT // 16):
        sl = pl.ds(j * 16, 16)
        buf_v[sl] = jnp.minimum(buf_v[sl], MAX_DEG - 1)

    @pl.when(c == 0)
    def _():
        pltpu.sync_copy(buf_v, dego_hbm.at[pl.ds(s * CPT, CPT)])

    @pl.when(c == 1)
    def _():
        pltpu.sync_copy(buf_v, degi_hbm.at[pl.ds(s * CPT, CPT)])


@functools.partial(
    pl.kernel,
    out_type=jax.ShapeDtypeStruct((N_NODES, NODE_DIM), jnp.float32),
    mesh=_mesh,
    scratch_types=[
        pltpu.VMEM((2, BLK), jnp.int32),
        pltpu.VMEM((2, BLK), jnp.int32),
        pltpu.VMEM((2, BLK, NODE_DIM), jnp.float32),
        pltpu.VMEM((2, BLK, NODE_DIM), jnp.float32),
        pltpu.VMEM((2, BLK, NODE_DIM), jnp.float32),
        pltpu.SemaphoreType.DMA,
        pltpu.SemaphoreType.DMA,
        pltpu.SemaphoreType.DMA,
        pltpu.SemaphoreType.DMA,
        pltpu.SemaphoreType.DMA,
        pltpu.SemaphoreType.DMA,
        pltpu.SemaphoreType.DMA,
        pltpu.SemaphoreType.DMA,
    ],
)
def _encode(x_hbm, dego_hbm, degi_hbm, zin_hbm, zout_hbm, out_hbm,
            degi_v, dego_v, acc_v, zi_v, zo_v,
            semd0, semd1, semx0, semx1, semg0, semg1, semw0, semw1):
    c = lax.axis_index("c")
    s = lax.axis_index("s")
    w = s * NC + c
    semd = (semd0, semd1)
    semx = (semx0, semx1)
    semg = (semg0, semg1)
    semw = (semw0, semw1)

    def issue_deg_x(k):
        p = k % 2
        b = k * NW + w

        @pl.when(b < NBLK)
        def _():
            base = b * BLK
            pltpu.async_copy(degi_hbm.at[pl.ds(base, BLK)], degi_v.at[p], semd[p])
            pltpu.async_copy(dego_hbm.at[pl.ds(base, BLK)], dego_v.at[p], semd[p])
            pltpu.async_copy(x_hbm.at[pl.ds(base, BLK), :], acc_v.at[p], semx[p])

    issue_deg_x(0)
    for k in range(MAXB):
        p = k % 2
        q = 1 - p
        b = k * NW + w

        @pl.when(b < NBLK)
        def _():
            base = b * BLK
            pltpu.make_async_copy(
                degi_hbm.at[pl.ds(base, BLK)], degi_v.at[p], semd[p]).wait()
            pltpu.make_async_copy(
                dego_hbm.at[pl.ds(base, BLK)], dego_v.at[p], semd[p]).wait()
            pltpu.async_copy(zin_hbm.at[degi_v.at[p]], zi_v.at[p], semg[p])
            pltpu.async_copy(zout_hbm.at[dego_v.at[p]], zo_v.at[p], semg[p])

        if k >= 1:
            bprev = (k - 1) * NW + w

            @pl.when(bprev < NBLK)
            def _():
                pltpu.make_async_copy(
                    acc_v.at[q], out_hbm.at[pl.ds(bprev * BLK, BLK), :],
                    semw[q]).wait()

        if k + 1 < MAXB:
            issue_deg_x(k + 1)

        @pl.when(b < NBLK)
        def _():
            base = b * BLK
            pltpu.make_async_copy(
                x_hbm.at[pl.ds(base, BLK), :], acc_v.at[p], semx[p]).wait()
            pltpu.make_async_copy(
                zin_hbm.at[degi_v.at[p]], zi_v.at[p], semg[p]).wait()
            pltpu.make_async_copy(
                zout_hbm.at[dego_v.at[p]], zo_v.at[p], semg[p]).wait()

            def row_add(r, carry):
                for jj in range(NODE_DIM // 16):
                    sl = pl.ds(jj * 16, 16)
                    acc_v[p, r, sl] = acc_v[p, r, sl] + zi_v[p, r, sl] + zo_v[p, r, sl]
                return carry

            lax.fori_loop(0, BLK, row_add, 0)
            pltpu.async_copy(acc_v.at[p], out_hbm.at[pl.ds(base, BLK), :], semw[p])

    blast = (MAXB - 1) * NW + w
    plast = (MAXB - 1) % 2

    @pl.when(blast < NBLK)
    def _():
        pltpu.make_async_copy(
            acc_v.at[plast], out_hbm.at[pl.ds(blast * BLK, BLK), :],
            semw[plast]).wait()


def kernel(x, edge_index, z_in, z_out):
    ei = edge_index.astype(jnp.int32).reshape(2, NS, RPT, ROWW)
    dego, degi = _degrees(ei)
    return _encode(x, dego, degi, z_in, z_out)
